# baseline (speedup 1.0000x reference)
import jax
import jax.numpy as jnp
from jax.experimental import pallas as pl

_K = 64
_D = 1024
_F = 8192
_T = 2048
_RT = 256


def _fused_body(x_ref, wenc_ref, wdec_ref, benc_ref, bdec_ref, out_ref):
    xb = (x_ref[...] - bdec_ref[...]).astype(jnp.bfloat16)
    pre = jax.lax.dot_general(
        xb, wenc_ref[...], (((1,), (1,)), ((), ())),
        preferred_element_type=jnp.float32)
    acts = jnp.maximum(pre + benc_ref[...], 0.0)
    ai = jax.lax.bitcast_convert_type(acts, jnp.int32)
    lo = jnp.zeros((acts.shape[0], 1), jnp.int32)

    def step(i, lo):
        cand = lo | (1 << (30 - i))
        cnt = jnp.sum((ai >= cand).astype(jnp.int32), axis=1, keepdims=True)
        return jnp.where(cnt >= _K, cand, lo)

    lo = jax.lax.fori_loop(0, 31, step, lo, unroll=True)
    code = jnp.where(ai >= lo, acts, 0.0).astype(jnp.bfloat16)
    out = jax.lax.dot_general(
        code, wdec_ref[...], (((1,), (1,)), ((), ())),
        preferred_element_type=jnp.float32)
    out_ref[...] = out + bdec_ref[...]


def _sae_forward(x, w_enc, b_enc, w_dec, b_dec):
    x2 = x.reshape(_T, _D)
    out = pl.pallas_call(
        _fused_body,
        grid=(_T // _RT,),
        in_specs=[
            pl.BlockSpec((_RT, _D), lambda i: (i, 0)),
            pl.BlockSpec((_F, _D), lambda i: (0, 0)),
            pl.BlockSpec((_D, _F), lambda i: (0, 0)),
            pl.BlockSpec((1, _F), lambda i: (0, 0)),
            pl.BlockSpec((1, _D), lambda i: (0, 0)),
        ],
        out_specs=pl.BlockSpec((_RT, _D), lambda i: (i, 0)),
        out_shape=jax.ShapeDtypeStruct((_T, _D), jnp.float32),
    )(x2, w_enc.astype(jnp.bfloat16), w_dec.astype(jnp.bfloat16),
      b_enc[None, :], b_dec[None, :])
    return out.reshape(x.shape)


def kernel(ln1_0, ln2_0, W_enc_attn_0, b_enc_attn_0, W_dec_attn_0, b_dec_attn_0,
           W_enc_mlp_0, b_enc_mlp_0, W_dec_mlp_0, b_dec_mlp_0):
    r_attn = _sae_forward(ln1_0, W_enc_attn_0, b_enc_attn_0, W_dec_attn_0, b_dec_attn_0)
    r_mlp = _sae_forward(ln2_0, W_enc_mlp_0, b_enc_mlp_0, W_dec_mlp_0, b_dec_mlp_0)
    return jnp.stack([r_attn, r_mlp], axis=0)


# speedup vs baseline: 1.5121x; 1.5121x over previous
import jax
import jax.numpy as jnp
from jax.experimental import pallas as pl

_K = 64
_D = 1024
_F = 8192
_T = 2048
_RT = 256


def _count_ge_i16(arr16, cand16):
    m = (arr16 >= cand16).astype(jnp.int16)
    w = m.shape[1]
    while w > 64:
        half = w // 2
        m = m[:, :half] + m[:, half:]
        w = half
    return jnp.sum(m.astype(jnp.int32), axis=1, keepdims=True)


def _topk_threshold_bits(ai):
    rows = ai.shape[0]
    h = (ai >> 16).astype(jnp.int16)

    lo_h = jnp.zeros((rows, 1), jnp.int32)

    def step_a(i, lo):
        cand = lo | (1 << (14 - i))
        cnt = _count_ge_i16(h, cand.astype(jnp.int16))
        return jnp.where(cnt >= _K, cand, lo)

    lo_h = jax.lax.fori_loop(0, 15, step_a, lo_h, unroll=True)
    t_h16 = lo_h.astype(jnp.int16)

    n_above = _count_ge_i16(h, (lo_h + 1).astype(jnp.int16))
    target = _K - n_above

    lo16s = ((ai & 0xFFFF) - 32768).astype(jnp.int16)
    rs = jnp.where(h == t_h16, lo16s, jnp.int16(-32768))

    lo_u = jnp.zeros((rows, 1), jnp.int32)

    def step_b(i, lo):
        cand_u = lo | (1 << (15 - i))
        cnt = _count_ge_i16(rs, (cand_u - 32768).astype(jnp.int16))
        return jnp.where(cnt >= target, cand_u, lo)

    lo_u = jax.lax.fori_loop(0, 16, step_b, lo_u, unroll=True)
    return (lo_h << 16) | lo_u


def _encode_body(x_ref, wenc_ref, benc_ref, bdec_ref, code_ref):
    xb = (x_ref[...] - bdec_ref[...]).astype(jnp.bfloat16)
    pre = jax.lax.dot_general(
        xb, wenc_ref[...], (((1,), (1,)), ((), ())),
        preferred_element_type=jnp.float32)
    acts = jnp.maximum(pre + benc_ref[...], 0.0)
    ai = jax.lax.bitcast_convert_type(acts, jnp.int32)
    thr = _topk_threshold_bits(ai)
    code_ref[...] = jnp.where(ai >= thr, acts, 0.0).astype(jnp.bfloat16)


def _decode_body(code_ref, wdec_ref, bdec_ref, out_ref):
    out = jax.lax.dot_general(
        code_ref[...], wdec_ref[...], (((1,), (1,)), ((), ())),
        preferred_element_type=jnp.float32)
    out_ref[...] = out + bdec_ref[...]


def _sae_forward(x, w_enc, b_enc, w_dec, b_dec):
    x2 = x.reshape(_T, _D)
    code = pl.pallas_call(
        _encode_body,
        grid=(_T // _RT,),
        in_specs=[
            pl.BlockSpec((_RT, _D), lambda i: (i, 0)),
            pl.BlockSpec((_F, _D), lambda i: (0, 0)),
            pl.BlockSpec((1, _F), lambda i: (0, 0)),
            pl.BlockSpec((1, _D), lambda i: (0, 0)),
        ],
        out_specs=pl.BlockSpec((_RT, _F), lambda i: (i, 0)),
        out_shape=jax.ShapeDtypeStruct((_T, _F), jnp.bfloat16),
    )(x2, w_enc.astype(jnp.bfloat16), b_enc[None, :], b_dec[None, :])
    out = pl.pallas_call(
        _decode_body,
        grid=(_T // _RT,),
        in_specs=[
            pl.BlockSpec((_RT, _F), lambda i: (i, 0)),
            pl.BlockSpec((_D, _F), lambda i: (0, 0)),
            pl.BlockSpec((1, _D), lambda i: (0, 0)),
        ],
        out_specs=pl.BlockSpec((_RT, _D), lambda i: (i, 0)),
        out_shape=jax.ShapeDtypeStruct((_T, _D), jnp.float32),
    )(code, w_dec.astype(jnp.bfloat16), b_dec[None, :])
    return out.reshape(x.shape)


def kernel(ln1_0, ln2_0, W_enc_attn_0, b_enc_attn_0, W_dec_attn_0, b_dec_attn_0,
           W_enc_mlp_0, b_enc_mlp_0, W_dec_mlp_0, b_dec_mlp_0):
    r_attn = _sae_forward(ln1_0, W_enc_attn_0, b_enc_attn_0, W_dec_attn_0, b_dec_attn_0)
    r_mlp = _sae_forward(ln2_0, W_enc_mlp_0, b_enc_mlp_0, W_dec_mlp_0, b_dec_mlp_0)
    return jnp.stack([r_attn, r_mlp], axis=0)
